# Initial kernel scaffold; baseline (speedup 1.0000x reference)
import functools

import jax
import jax.numpy as jnp
from jax import lax
from jax.experimental import pallas as pl
from jax.experimental.pallas import tpu as pltpu
from jax.experimental.pallas import tpu_sc as plsc

_N = 10000
_E = 320000
_D = 128

_NC = 2
_NS = 16
_NW = _NC * _NS
_CHUNK = 128
_NCHUNK = 79
_EPW = _NCHUNK * _CHUNK
_ROWS_PER_TILE = _N // _NS
_ZROWS = 125


def _mm_bias_body(x_ref, w_ref, b_ref, o_ref):
    o_ref[...] = (
        jnp.dot(x_ref[...], w_ref[...], preferred_element_type=jnp.float32)
        + b_ref[...]
    )


def _mm_bias(x, w, b):
    return pl.pallas_call(
        _mm_bias_body,
        out_shape=jax.ShapeDtypeStruct((x.shape[0], w.shape[1]), jnp.float32),
    )(x, w, b.reshape(1, -1))


def _combine_relu_mm_body(p0_ref, p1_ref, w_ref, b_ref, o_ref):
    h = jnp.maximum(p0_ref[...] + p1_ref[...], 0.0)
    o_ref[...] = (
        jnp.dot(h, w_ref[...], preferred_element_type=jnp.float32) + b_ref[...]
    )


def _combine_relu_mm(p0, p1, w, b):
    return pl.pallas_call(
        _combine_relu_mm_body,
        out_shape=jax.ShapeDtypeStruct((p0.shape[0], w.shape[1]), jnp.float32),
    )(p0, p1, w, b.reshape(1, -1))


def _add_body(p0_ref, p1_ref, o_ref):
    o_ref[...] = p0_ref[...] + p1_ref[...]


def _combine_add(p0, p1):
    return pl.pallas_call(
        _add_body,
        out_shape=jax.ShapeDtypeStruct(p0.shape, jnp.float32),
    )(p0, p1)


def _sc_body(sup_hbm, src_hbm, dst_hbm, w_hbm, out_hbm,
             src_v, dst_v, w_v, rows_v, acc, sem):
    c = lax.axis_index("c")
    s = lax.axis_index("s")
    wid = s * _NC + c

    pltpu.sync_copy(src_hbm.at[wid], src_v)
    pltpu.sync_copy(dst_hbm.at[wid], dst_v)
    pltpu.sync_copy(w_hbm.at[wid], w_v)

    def _zrow(r, carry):
        for cc in range(_D // 16):
            rows_v[r, pl.ds(cc * 16, 16)] = jnp.zeros((16,), jnp.float32)
        return carry
    lax.fori_loop(0, _ZROWS, _zrow, 0)
    base = s * _ROWS_PER_TILE
    for z in range(_ROWS_PER_TILE // _ZROWS):
        pltpu.sync_copy(
            rows_v.at[pl.ds(0, _ZROWS)],
            acc.at[pl.ds(base + z * _ZROWS, _ZROWS)],
        )
    plsc.subcore_barrier()

    def _chunk(j, carry):
        pltpu.async_copy(sup_hbm.at[src_v.at[j]], rows_v, sem).wait()

        def _row(r, rcarry):
            widx = jnp.full((16,), j, jnp.int32)
            ridx = jnp.full((16,), r, jnp.int32)
            wvec = plsc.load_gather(w_v, [widx, ridx])
            for cc in range(_D // 16):
                sl = pl.ds(cc * 16, 16)
                rows_v[r, sl] = rows_v[r, sl] * wvec
            return rcarry

        lax.fori_loop(0, _CHUNK, _row, 0)
        pltpu.sync_copy(rows_v, acc.at[dst_v.at[j]], add=True)
        return carry

    lax.fori_loop(0, _NCHUNK, _chunk, 0)
    plsc.subcore_barrier()

    pltpu.sync_copy(
        acc.at[pl.ds(base, _ROWS_PER_TILE)],
        out_hbm.at[c, pl.ds(base, _ROWS_PER_TILE)],
    )


_sc_scatter = functools.partial(
    pl.kernel,
    mesh=plsc.VectorSubcoreMesh(core_axis_name="c", subcore_axis_name="s"),
    out_type=jax.ShapeDtypeStruct((_NC, _N, _D), jnp.float32),
    scratch_types=[
        pltpu.VMEM((_NCHUNK, _CHUNK), jnp.int32),
        pltpu.VMEM((_NCHUNK, _CHUNK), jnp.int32),
        pltpu.VMEM((_NCHUNK, _CHUNK), jnp.float32),
        pltpu.VMEM((_CHUNK, _D), jnp.float32),
        pltpu.VMEM_SHARED((_N, _D), jnp.float32),
        pltpu.SemaphoreType.DMA,
    ],
)(_sc_body)


@jax.jit
def kernel(inp, edge_index, edge_weight, W1, b1, W2, b2):
    src = edge_index[0]
    dst = edge_index[1]
    pad = _NW * _EPW - _E
    srcp = jnp.concatenate(
        [src, jnp.zeros((pad,), jnp.int32)]).reshape(_NW, _NCHUNK, _CHUNK)
    dstp = jnp.concatenate(
        [dst, jnp.zeros((pad,), jnp.int32)]).reshape(_NW, _NCHUNK, _CHUNK)
    wp = jnp.concatenate(
        [edge_weight, jnp.zeros((pad,), jnp.float32)]
    ).reshape(_NW, _NCHUNK, _CHUNK)

    s1 = _mm_bias(inp, W1, b1)
    p = _sc_scatter(s1, srcp, dstp, wp)
    s2 = _combine_relu_mm(p[0], p[1], W2, b2)
    q = _sc_scatter(s2, srcp, dstp, wp)
    return _combine_add(q[0], q[1])


# speedup vs baseline: 4.2910x; 4.2910x over previous
import functools

import jax
import jax.numpy as jnp
from jax import lax
from jax.experimental import pallas as pl
from jax.experimental.pallas import tpu as pltpu
from jax.experimental.pallas import tpu_sc as plsc

_N = 10000
_E = 320000
_D = 128

_NC = 2
_NS = 16
_NW = _NC * _NS
_CHUNK = 128
_NCHUNK = 79
_EPW = _NCHUNK * _CHUNK
_NPAD = 10240
_ROWS_PER_TILE = _NPAD // _NS
_ZROWS = _CHUNK


def _mm_bias_body(x_ref, w_ref, b_ref, o_ref):
    o_ref[...] = (
        jnp.dot(x_ref[...], w_ref[...], preferred_element_type=jnp.float32)
        + b_ref[...]
    )


def _mm_bias(x, w, b):
    return pl.pallas_call(
        _mm_bias_body,
        out_shape=jax.ShapeDtypeStruct((x.shape[0], w.shape[1]), jnp.float32),
    )(x, w, b.reshape(1, -1))


def _combine_relu_mm_body(p0_ref, p1_ref, w_ref, b_ref, o_ref):
    h = jnp.maximum(p0_ref[...] + p1_ref[...], 0.0)
    o_ref[...] = (
        jnp.dot(h, w_ref[...], preferred_element_type=jnp.float32) + b_ref[...]
    )


def _combine_relu_mm(p0, p1, w, b):
    return pl.pallas_call(
        _combine_relu_mm_body,
        out_shape=jax.ShapeDtypeStruct((p0.shape[0], w.shape[1]), jnp.float32),
    )(p0, p1, w, b.reshape(1, -1))


def _add_body(p0_ref, p1_ref, o_ref):
    o_ref[...] = p0_ref[...] + p1_ref[...]


def _combine_add(p0, p1):
    return pl.pallas_call(
        _add_body,
        out_shape=jax.ShapeDtypeStruct(p0.shape, jnp.float32),
    )(p0, p1)


def _bcast_lane(vec16, k):
    idx = jnp.full((16, 1), k, jnp.int32)
    dnums = lax.GatherDimensionNumbers(
        offset_dims=(), collapsed_slice_dims=(0,), start_index_map=(0,))
    return lax.gather(
        vec16, idx, dnums, (1,),
        mode=lax.GatherScatterMode.PROMISE_IN_BOUNDS)


def _sc_body(sup_hbm, src_hbm, dst_hbm, w_hbm, out_hbm,
             src_v, dst_v, w_v, rows_v, acc, sem):
    c = lax.axis_index("c")
    s = lax.axis_index("s")
    wid = s * _NC + c

    pltpu.sync_copy(src_hbm.at[wid], src_v)
    pltpu.sync_copy(dst_hbm.at[wid], dst_v)
    pltpu.sync_copy(w_hbm.at[wid], w_v)

    def _zrow(r, carry):
        for cc in range(_D // 16):
            rows_v[r, pl.ds(cc * 16, 16)] = jnp.zeros((16,), jnp.float32)
        return carry
    lax.fori_loop(0, _ZROWS, _zrow, 0, unroll=4)
    base = s * _ROWS_PER_TILE
    for z in range(_ROWS_PER_TILE // _ZROWS):
        pltpu.sync_copy(
            rows_v,
            acc.at[pl.ds(base + z * _ZROWS, _ZROWS)],
        )
    plsc.subcore_barrier()

    def _chunk(j, carry):
        pltpu.async_copy(sup_hbm.at[src_v.at[j]], rows_v, sem).wait()

        def _grp(g, gcarry):
            wv = w_v[j, pl.ds(g * 16, 16)]
            for k in range(16):
                r = g * 16 + k
                wvec = _bcast_lane(wv, k)
                for cc in range(_D // 16):
                    sl = pl.ds(cc * 16, 16)
                    rows_v[r, sl] = rows_v[r, sl] * wvec
            return gcarry

        lax.fori_loop(0, _CHUNK // 16, _grp, 0)
        pltpu.sync_copy(rows_v, acc.at[dst_v.at[j]], add=True)
        return carry

    lax.fori_loop(0, _NCHUNK, _chunk, 0)
    plsc.subcore_barrier()

    pltpu.sync_copy(
        acc.at[pl.ds(base, _ROWS_PER_TILE)],
        out_hbm.at[c, pl.ds(base, _ROWS_PER_TILE)],
    )


_sc_scatter = functools.partial(
    pl.kernel,
    mesh=plsc.VectorSubcoreMesh(
        core_axis_name="c", subcore_axis_name="s",
        num_cores=_NC, num_subcores=_NS),
    out_type=jax.ShapeDtypeStruct((_NC, _NPAD, _D), jnp.float32),
    scratch_types=[
        pltpu.VMEM((_NCHUNK, _CHUNK), jnp.int32),
        pltpu.VMEM((_NCHUNK, _CHUNK), jnp.int32),
        pltpu.VMEM((_NCHUNK, _CHUNK), jnp.float32),
        pltpu.VMEM((_CHUNK, _D), jnp.float32),
        pltpu.VMEM_SHARED((_NPAD, _D), jnp.float32),
        pltpu.SemaphoreType.DMA,
    ],
)(_sc_body)


@jax.jit
def kernel(inp, edge_index, edge_weight, W1, b1, W2, b2):
    src = edge_index[0]
    dst = edge_index[1]
    pad = _NW * _EPW - _E
    srcp = jnp.concatenate(
        [src, jnp.zeros((pad,), jnp.int32)]).reshape(_NW, _NCHUNK, _CHUNK)
    dstp = jnp.concatenate(
        [dst, jnp.zeros((pad,), jnp.int32)]).reshape(_NW, _NCHUNK, _CHUNK)
    wp = jnp.concatenate(
        [edge_weight, jnp.zeros((pad,), jnp.float32)]
    ).reshape(_NW, _NCHUNK, _CHUNK)

    s1 = _mm_bias(inp, W1, b1)
    p = _sc_scatter(s1, srcp, dstp, wp)
    s2 = _combine_relu_mm(p[0, :_N], p[1, :_N], W2, b2)
    q = _sc_scatter(s2, srcp, dstp, wp)
    return _combine_add(q[0, :_N], q[1, :_N])
